# baseline (speedup 1.0000x reference)
import functools

import jax
import jax.numpy as jnp
from jax import lax
from jax.experimental import pallas as pl
from jax.experimental.pallas import tpu as pltpu
from jax.experimental.pallas import tpu_sc as plsc

C_IN = 100000
D_MODEL = 64
W32 = D_MODEL // 2
BATCH = 4096
SEQ = 200
ROW_WORDS = SEQ * W32
STAGE_MINOR = 128
STAGE_ROWS_PER_CHUNK = ROW_WORDS // STAGE_MINOR

_info = plsc.get_sparse_core_info()
NC = _info.num_cores
NS = _info.num_subcores
NW = NC * NS
ROWS_PER_W = BATCH // NW
NBUF = 2


def _gather_kernel(x_hbm, w_hbm, stage_hbm, idx_v0, idx_v1, rows_v,
                   sem_idx, sem_g, sem_w):
    idx_v = (idx_v0, idx_v1)
    wid = lax.axis_index("s") * NC + lax.axis_index("c")
    base = wid * ROWS_PER_W

    def stage_slot(r):
        return stage_hbm.at[r, :, pl.ds(0, W32)]

    for b in range(NBUF):
        pltpu.async_copy(x_hbm.at[base + b], idx_v[b], sem_idx.at[b])

    def super_body(s, carry):
        for b in range(NBUF):
            g = s * NBUF + b
            r = base + g
            src = rows_v.at[b]
            @pl.when(s > 0)
            def _():
                pltpu.make_async_copy(
                    src, stage_slot(r - NBUF), sem_w.at[b]).wait()
            pltpu.make_async_copy(
                x_hbm.at[r], idx_v[b], sem_idx.at[b]).wait()
            pltpu.async_copy(w_hbm.at[idx_v[b]], rows_v.at[b],
                             sem_g.at[b]).wait()
            @pl.when(g + NBUF < ROWS_PER_W)
            def _():
                pltpu.async_copy(
                    x_hbm.at[r + NBUF], idx_v[b], sem_idx.at[b])
            pltpu.async_copy(src, stage_slot(r), sem_w.at[b])
        return carry

    lax.fori_loop(0, ROWS_PER_W // NBUF, super_body, 0)

    for b in range(NBUF):
        r = base + ROWS_PER_W - NBUF + b
        pltpu.make_async_copy(rows_v.at[b], stage_slot(r), sem_w.at[b]).wait()


@jax.jit
def _embed(x, W):
    w_bits = lax.bitcast_convert_type(
        W.astype(jnp.bfloat16).reshape(C_IN, W32, 2), jnp.float32)
    mesh = plsc.VectorSubcoreMesh(core_axis_name="c", subcore_axis_name="s")
    gather = functools.partial(
        pl.kernel,
        mesh=mesh,
        out_type=jax.ShapeDtypeStruct(
            (BATCH, SEQ, STAGE_MINOR), jnp.float32),
        scratch_types=[
            pltpu.VMEM((SEQ,), jnp.int32),
            pltpu.VMEM((SEQ,), jnp.int32),
            pltpu.VMEM((NBUF, SEQ, W32), jnp.float32),
            pltpu.SemaphoreType.DMA((NBUF,)),
            pltpu.SemaphoreType.DMA((NBUF,)),
            pltpu.SemaphoreType.DMA((NBUF,)),
        ],
        compiler_params=pltpu.CompilerParams(use_tc_tiling_on_sc=False),
    )(_gather_kernel)
    stage = gather(x, w_bits)
    out_bf = lax.bitcast_convert_type(stage[:, :, :W32], jnp.bfloat16)
    return out_bf.reshape(BATCH, SEQ, D_MODEL).astype(jnp.float32)


def kernel(x, W):
    return _embed(x, W)


# speedup vs baseline: 2.9479x; 2.9479x over previous
import functools

import jax
import jax.numpy as jnp
from jax import lax
from jax.experimental import pallas as pl
from jax.experimental.pallas import tpu as pltpu
from jax.experimental.pallas import tpu_sc as plsc

C_IN = 100000
D_MODEL = 64
W32 = D_MODEL
BATCH = 4096
SEQ = 200
ROW_WORDS = SEQ * W32
STAGE_MINOR = 128
STAGE_ROWS_PER_CHUNK = ROW_WORDS // STAGE_MINOR

_info = plsc.get_sparse_core_info()
NC = _info.num_cores
NS = _info.num_subcores
NW = NC * NS
ROWS_PER_W = BATCH // NW
NBUF = 2


def _gather_kernel(x_hbm, w_hbm, stage_hbm, idx_v0, idx_v1, rows_v,
                   sem_idx, sem_g, sem_w):
    idx_v = (idx_v0, idx_v1)
    wid = lax.axis_index("s") * NC + lax.axis_index("c")
    base = wid * ROWS_PER_W

    def stage_slot(r):
        return stage_hbm.at[r, :, pl.ds(0, D_MODEL)]

    for b in range(NBUF):
        pltpu.async_copy(x_hbm.at[base + b], idx_v[b], sem_idx.at[b])

    def super_body(s, carry):
        for b in range(NBUF):
            g = s * NBUF + b
            r = base + g
            src = rows_v.at[b]
            @pl.when(s > 0)
            def _():
                pltpu.make_async_copy(
                    src, stage_slot(r - NBUF), sem_w.at[b]).wait()
            pltpu.make_async_copy(
                x_hbm.at[r], idx_v[b], sem_idx.at[b]).wait()
            pltpu.async_copy(w_hbm.at[idx_v[b]], rows_v.at[b],
                             sem_g.at[b]).wait()
            @pl.when(g + NBUF < ROWS_PER_W)
            def _():
                pltpu.async_copy(
                    x_hbm.at[r + NBUF], idx_v[b], sem_idx.at[b])
            pltpu.async_copy(src, stage_slot(r), sem_w.at[b])
        return carry

    lax.fori_loop(0, ROWS_PER_W // NBUF, super_body, 0)

    for b in range(NBUF):
        r = base + ROWS_PER_W - NBUF + b
        pltpu.make_async_copy(rows_v.at[b], stage_slot(r), sem_w.at[b]).wait()


@jax.jit
def _embed(x, W):
    mesh = plsc.VectorSubcoreMesh(core_axis_name="c", subcore_axis_name="s")
    gather = functools.partial(
        pl.kernel,
        mesh=mesh,
        out_type=jax.ShapeDtypeStruct(
            (BATCH, SEQ, STAGE_MINOR), jnp.float32),
        scratch_types=[
            pltpu.VMEM((SEQ,), jnp.int32),
            pltpu.VMEM((SEQ,), jnp.int32),
            pltpu.VMEM((NBUF, SEQ, D_MODEL), jnp.float32),
            pltpu.SemaphoreType.DMA((NBUF,)),
            pltpu.SemaphoreType.DMA((NBUF,)),
            pltpu.SemaphoreType.DMA((NBUF,)),
        ],
        compiler_params=pltpu.CompilerParams(use_tc_tiling_on_sc=False),
    )(_gather_kernel)
    stage = gather(x, W)
    return stage[:, :, :D_MODEL]


def kernel(x, W):
    return _embed(x, W)
